# baseline (speedup 1.0000x reference)
import math

import jax
import jax.numpy as jnp
from jax import lax
from jax.experimental import pallas as pl
from jax.experimental.pallas import tpu as pltpu
from jax.experimental.pallas import tpu_sc as plsc

N = 10000
E = 320000
D = 128

NC = 2
NS = 16
NW = NC * NS

CHUNK = 128
NCHUNK = 80
EPT = NCHUNK * CHUNK
E_PAD = EPT * NW

N_PAD = 10112
ROWS_PER_TILE = N_PAD // NS

INV_SQRT = float(1.0 / math.sqrt(1.0 + 1e-5))

ROW_BLK = 2000


def _sc_body(y2_hbm, zeros_hbm, col_hbm, row_hbm, out_hbm,
             col_v, row_v, buf, agg, sem):
    c = lax.axis_index("c")
    s = lax.axis_index("s")

    start = jnp.where(c == 0, NCHUNK, 0)
    pltpu.sync_copy(col_hbm.at[s].at[pl.ds(start, NCHUNK)], col_v)
    pltpu.sync_copy(row_hbm.at[s].at[pl.ds(start, NCHUNK)], row_v)

    pltpu.sync_copy(zeros_hbm.at[pl.ds(s * ROWS_PER_TILE, ROWS_PER_TILE)],
                    agg.at[pl.ds(s * ROWS_PER_TILE, ROWS_PER_TILE)])
    plsc.subcore_barrier()

    def body(j, _):
        pltpu.async_copy(y2_hbm.at[col_v.at[j]], buf, sem).wait()
        pltpu.sync_copy(buf, agg.at[row_v.at[j]], add=True)
        return 0

    lax.fori_loop(0, NCHUNK, body, 0)

    plsc.subcore_barrier()
    pltpu.sync_copy(agg.at[pl.ds(s * ROWS_PER_TILE, ROWS_PER_TILE)],
                    out_hbm.at[c].at[pl.ds(s * ROWS_PER_TILE, ROWS_PER_TILE)])


_sc_agg = pl.kernel(
    _sc_body,
    out_type=jax.ShapeDtypeStruct((NC, N_PAD, D), jnp.float32),
    mesh=plsc.VectorSubcoreMesh(core_axis_name="c", subcore_axis_name="s"),
    scratch_types=[
        pltpu.VMEM((NCHUNK, CHUNK), jnp.int32),
        pltpu.VMEM((NCHUNK, CHUNK), jnp.int32),
        pltpu.VMEM((CHUNK, D), jnp.float32),
        pltpu.VMEM_SHARED((N_PAD, D), jnp.float32),
        pltpu.SemaphoreType.DMA,
    ],
)


def _mm2_body(x_ref, w1_ref, w2_ref, y1_ref, y2_ref):
    xb = x_ref[...]
    y1_ref[...] = jnp.dot(xb, w1_ref[...], preferred_element_type=jnp.float32)
    y2_ref[...] = jnp.dot(xb, w2_ref[...], preferred_element_type=jnp.float32)


def _tc_mm2(x, w1, w2):
    return pl.pallas_call(
        _mm2_body,
        grid=(N // ROW_BLK,),
        in_specs=[
            pl.BlockSpec((ROW_BLK, D), lambda i: (i, 0)),
            pl.BlockSpec((D, D), lambda i: (0, 0)),
            pl.BlockSpec((D, D), lambda i: (0, 0)),
        ],
        out_specs=[pl.BlockSpec((ROW_BLK, D), lambda i: (i, 0))] * 2,
        out_shape=[jax.ShapeDtypeStruct((N, D), jnp.float32)] * 2,
    )(x, w1, w2)


def _fuse_mm2_body(y1_ref, a0_ref, a1_ref, g_ref, b_ref, w1_ref, w2_ref,
                   o1_ref, o2_ref):
    h = (y1_ref[...] + a0_ref[...] + a1_ref[...]) * (g_ref[...] * INV_SQRT)
    h = jnp.maximum(h + b_ref[...], 0.0)
    o1_ref[...] = jnp.dot(h, w1_ref[...], preferred_element_type=jnp.float32)
    o2_ref[...] = jnp.dot(h, w2_ref[...], preferred_element_type=jnp.float32)


def _tc_fuse_mm2(y1, a0, a1, gamma, beta, w1, w2):
    return pl.pallas_call(
        _fuse_mm2_body,
        grid=(N // ROW_BLK,),
        in_specs=[
            pl.BlockSpec((ROW_BLK, D), lambda i: (i, 0)),
            pl.BlockSpec((ROW_BLK, D), lambda i: (i, 0)),
            pl.BlockSpec((ROW_BLK, D), lambda i: (i, 0)),
            pl.BlockSpec((1, D), lambda i: (0, 0)),
            pl.BlockSpec((1, D), lambda i: (0, 0)),
            pl.BlockSpec((D, D), lambda i: (0, 0)),
            pl.BlockSpec((D, D), lambda i: (0, 0)),
        ],
        out_specs=[pl.BlockSpec((ROW_BLK, D), lambda i: (i, 0))] * 2,
        out_shape=[jax.ShapeDtypeStruct((N, D), jnp.float32)] * 2,
    )(y1, a0, a1, gamma, beta, w1, w2)


def _final_body(y1_ref, a0_ref, a1_ref, g_ref, b_ref, o_ref):
    h = (y1_ref[...] + a0_ref[...] + a1_ref[...]) * (g_ref[...] * INV_SQRT)
    o_ref[...] = jnp.maximum(h + b_ref[...], 0.0)


def _tc_final(y1, a0, a1, gamma, beta):
    return pl.pallas_call(
        _final_body,
        grid=(N // ROW_BLK,),
        in_specs=[
            pl.BlockSpec((ROW_BLK, D), lambda i: (i, 0)),
            pl.BlockSpec((ROW_BLK, D), lambda i: (i, 0)),
            pl.BlockSpec((ROW_BLK, D), lambda i: (i, 0)),
            pl.BlockSpec((1, D), lambda i: (0, 0)),
            pl.BlockSpec((1, D), lambda i: (0, 0)),
        ],
        out_specs=pl.BlockSpec((ROW_BLK, D), lambda i: (i, 0)),
        out_shape=jax.ShapeDtypeStruct((N, D), jnp.float32),
    )(y1, a0, a1, gamma, beta)


@jax.jit
def kernel(x, local_edge_index, W1_0, W2_0, gamma0, beta0,
           W1_1, W2_1, gamma1, beta1):
    row = local_edge_index[0]
    col = local_edge_index[1]
    pad = E_PAD - E
    row3 = jnp.concatenate(
        [row, jnp.full((pad,), N, jnp.int32)]).reshape(NS, 2 * NCHUNK, CHUNK)
    col3 = jnp.concatenate(
        [col, jnp.zeros((pad,), jnp.int32)]).reshape(NS, 2 * NCHUNK, CHUNK)
    zeros = jnp.zeros((N_PAD, D), jnp.float32)
    g0 = gamma0.reshape(1, D)
    b0 = beta0.reshape(1, D)
    g1 = gamma1.reshape(1, D)
    b1 = beta1.reshape(1, D)

    y1_0, y2_0 = _tc_mm2(x, W1_0, W2_0)
    aggp0 = _sc_agg(y2_0, zeros, col3, row3)
    y1_1, y2_1 = _tc_fuse_mm2(y1_0, aggp0[0, :N], aggp0[1, :N],
                              g0, b0, W1_1, W2_1)
    aggp1 = _sc_agg(y2_1, zeros, col3, row3)
    return _tc_final(y1_1, aggp1[0, :N], aggp1[1, :N], g1, b1)


# speedup vs baseline: 2.8555x; 2.8555x over previous
import math

import jax
import jax.numpy as jnp
from jax import lax
from jax.experimental import pallas as pl
from jax.experimental.pallas import tpu as pltpu
from jax.experimental.pallas import tpu_sc as plsc

N = 10000
E = 320000
D = 128

NC = 2
NS = 16
NW = NC * NS

CHUNK = 128
NCHUNK = 80
EPT = NCHUNK * CHUNK
E_PAD = EPT * NW

N_PAD = 10112
ROWS_PER_TILE = N_PAD // NS

INV_SQRT = float(1.0 / math.sqrt(1.0 + 1e-5))

ROW_BLK = 2000


def _sc_body(y2_hbm, zeros_hbm, col_hbm, row_hbm, out_hbm,
             col_v, row_v, buf, agg, sem):
    c = lax.axis_index("c")
    s = lax.axis_index("s")

    start = jnp.where(c == 0, NCHUNK, 0)
    pltpu.sync_copy(col_hbm.at[s].at[pl.ds(start, NCHUNK)], col_v)
    pltpu.sync_copy(row_hbm.at[s].at[pl.ds(start, NCHUNK)], row_v)

    pltpu.sync_copy(zeros_hbm.at[pl.ds(s * ROWS_PER_TILE, ROWS_PER_TILE)],
                    agg.at[pl.ds(s * ROWS_PER_TILE, ROWS_PER_TILE)])
    plsc.subcore_barrier()

    def body(j, _):
        pltpu.async_copy(y2_hbm.at[col_v.at[j]], buf, sem).wait()
        pltpu.sync_copy(buf, agg.at[row_v.at[j]], add=True)
        return 0

    lax.fori_loop(0, NCHUNK, body, 0)

    plsc.subcore_barrier()
    pltpu.sync_copy(agg.at[pl.ds(s * ROWS_PER_TILE, ROWS_PER_TILE)],
                    out_hbm.at[c].at[pl.ds(s * ROWS_PER_TILE, ROWS_PER_TILE)])


_sc_agg = pl.kernel(
    _sc_body,
    out_type=jax.ShapeDtypeStruct((NC, N_PAD, D), jnp.float32),
    mesh=plsc.VectorSubcoreMesh(core_axis_name="c", subcore_axis_name="s"),
    scratch_types=[
        pltpu.VMEM((NCHUNK, CHUNK), jnp.int32),
        pltpu.VMEM((NCHUNK, CHUNK), jnp.int32),
        pltpu.VMEM((CHUNK, D), jnp.float32),
        pltpu.VMEM_SHARED((N_PAD, D), jnp.float32),
        pltpu.SemaphoreType.DMA,
    ],
)


def _mm2_body(x_ref, w1_ref, w2_ref, y1_ref, y2_ref):
    xb = x_ref[...]
    y1_ref[...] = jnp.dot(xb, w1_ref[...], preferred_element_type=jnp.float32)
    y2_ref[...] = jnp.dot(xb, w2_ref[...], preferred_element_type=jnp.float32)


def _tc_mm2(x, w1, w2):
    return pl.pallas_call(
        _mm2_body,
        grid=(N // ROW_BLK,),
        in_specs=[
            pl.BlockSpec((ROW_BLK, D), lambda i: (i, 0)),
            pl.BlockSpec((D, D), lambda i: (0, 0)),
            pl.BlockSpec((D, D), lambda i: (0, 0)),
        ],
        out_specs=[pl.BlockSpec((ROW_BLK, D), lambda i: (i, 0))] * 2,
        out_shape=[jax.ShapeDtypeStruct((N, D), jnp.float32)] * 2,
    )(x, w1, w2)


def _fuse_mm2_body(y1_ref, a0_ref, a1_ref, g_ref, b_ref, w1_ref, w2_ref,
                   o1_ref, o2_ref):
    h = (y1_ref[...] + a0_ref[...] + a1_ref[...]) * (g_ref[...] * INV_SQRT)
    h = jnp.maximum(h + b_ref[...], 0.0)
    o1_ref[...] = jnp.dot(h, w1_ref[...], preferred_element_type=jnp.float32)
    o2_ref[...] = jnp.dot(h, w2_ref[...], preferred_element_type=jnp.float32)


def _tc_fuse_mm2(y1, a0, a1, gamma, beta, w1, w2):
    return pl.pallas_call(
        _fuse_mm2_body,
        grid=(N // ROW_BLK,),
        in_specs=[
            pl.BlockSpec((ROW_BLK, D), lambda i: (i, 0)),
            pl.BlockSpec((ROW_BLK, D), lambda i: (i, 0)),
            pl.BlockSpec((ROW_BLK, D), lambda i: (i, 0)),
            pl.BlockSpec((1, D), lambda i: (0, 0)),
            pl.BlockSpec((1, D), lambda i: (0, 0)),
            pl.BlockSpec((D, D), lambda i: (0, 0)),
            pl.BlockSpec((D, D), lambda i: (0, 0)),
        ],
        out_specs=[pl.BlockSpec((ROW_BLK, D), lambda i: (i, 0))] * 2,
        out_shape=[jax.ShapeDtypeStruct((N, D), jnp.float32)] * 2,
    )(y1, a0, a1, gamma, beta, w1, w2)


def _final_body(y1_ref, a0_ref, a1_ref, g_ref, b_ref, o_ref):
    h = (y1_ref[...] + a0_ref[...] + a1_ref[...]) * (g_ref[...] * INV_SQRT)
    o_ref[...] = jnp.maximum(h + b_ref[...], 0.0)


def _tc_final(y1, a0, a1, gamma, beta):
    return pl.pallas_call(
        _final_body,
        grid=(N // ROW_BLK,),
        in_specs=[
            pl.BlockSpec((ROW_BLK, D), lambda i: (i, 0)),
            pl.BlockSpec((ROW_BLK, D), lambda i: (i, 0)),
            pl.BlockSpec((ROW_BLK, D), lambda i: (i, 0)),
            pl.BlockSpec((1, D), lambda i: (0, 0)),
            pl.BlockSpec((1, D), lambda i: (0, 0)),
        ],
        out_specs=pl.BlockSpec((ROW_BLK, D), lambda i: (i, 0)),
        out_shape=jax.ShapeDtypeStruct((N, D), jnp.float32),
    )(y1, a0, a1, gamma, beta)


@jax.jit
def kernel(x, local_edge_index, W1_0, W2_0, gamma0, beta0,
           W1_1, W2_1, gamma1, beta1):
    row = local_edge_index[0]
    col = local_edge_index[1]
    pad = E_PAD - E
    row3 = jnp.concatenate(
        [row, jnp.full((pad,), N, jnp.int32)]).reshape(NS, 2 * NCHUNK, CHUNK)
    col3 = jnp.concatenate(
        [col, jnp.arange(pad, dtype=jnp.int32) % N]).reshape(
            NS, 2 * NCHUNK, CHUNK)
    zeros = jnp.zeros((N_PAD, D), jnp.float32)
    g0 = gamma0.reshape(1, D)
    b0 = beta0.reshape(1, D)
    g1 = gamma1.reshape(1, D)
    b1 = beta1.reshape(1, D)

    y1_0, y2_0 = _tc_mm2(x, W1_0, W2_0)
    aggp0 = _sc_agg(y2_0, zeros, col3, row3)
    y1_1, y2_1 = _tc_fuse_mm2(y1_0, aggp0[0, :N], aggp0[1, :N],
                              g0, b0, W1_1, W2_1)
    aggp1 = _sc_agg(y2_1, zeros, col3, row3)
    return _tc_final(y1_1, aggp1[0, :N], aggp1[1, :N], g1, b1)
